# baseline (speedup 1.0000x reference)
import jax
import jax.numpy as jnp
from jax import lax
from jax.experimental import pallas as pl
from jax.experimental.pallas import tpu as pltpu
from jax.experimental.pallas import tpu_sc as plsc

VOCAB = 100000
DIM = 64
DIMH = DIM // 2
B = 4096
KN = 20
K = 2 * KN + 1
NC = 2
NS = 16
NW = NC * NS
BW = B // NW
NG = BW // 16
GK = 5
NGRP = (K - 1) // GK


def _iota16():
    return lax.broadcasted_iota(jnp.int32, (16,), 0)


def _sc_body(pair_u, pair_v, pos_u, neg_u, mask_pos, mask_neg,
             u_table, v_table, out,
             idxbuf, maskbuf, pvbuf, ps, ns, mp, mn, vrows, mdotbuf,
             abuf, bbuf, semv, semp, asem, bsem):
    wid = lax.axis_index("s") * NC + lax.axis_index("c")
    base = wid * BW
    bsl = pl.ds(base, BW)

    pltpu.sync_copy(pair_u.at[bsl], idxbuf.at[pl.ds(0, BW)])
    pltpu.sync_copy(pair_v.at[bsl], pvbuf)
    cpv = pltpu.async_copy(v_table.at[pvbuf], vrows, semv)
    cpp = pltpu.async_copy(u_table.at[idxbuf.at[pl.ds(0, BW)]],
                           abuf.at[pl.ds(0, BW)], semp)
    pltpu.sync_copy(pos_u.at[bsl], ps)
    pltpu.sync_copy(neg_u.at[bsl], ns)
    pltpu.sync_copy(mask_pos.at[bsl], mp)
    pltpu.sync_copy(mask_neg.at[bsl], mn)

    for g in range(NG):
        maskbuf[0, pl.ds(g * 16, 16)] = jnp.full((16,), 1.0, jnp.float32)

    def tbody(k, c):
        kv = jnp.full((16,), 0, jnp.int32) + k
        for g in range(NG):
            biota = _iota16() + g * 16
            gsl = pl.ds(g * 16, 16)
            idxbuf[pl.ds((1 + k) * BW + g * 16, 16)] = plsc.load_gather(
                ps, [biota, kv])
            idxbuf[pl.ds((1 + KN + k) * BW + g * 16, 16)] = plsc.load_gather(
                ns, [biota, kv])
            maskbuf[1 + k, gsl] = plsc.load_gather(mp, [biota, kv])
            maskbuf[1 + KN + k, gsl] = plsc.load_gather(mn, [biota, kv])
        return c

    lax.fori_loop(0, KN, tbody, 0)

    def start_group(kbase, buf, sem):
        pltpu.async_copy(u_table.at[idxbuf.at[pl.ds(kbase * BW, GK * BW)]],
                         buf, sem)

    def wait_group(buf, sem):
        pltpu.make_async_copy(u_table.at[idxbuf.at[pl.ds(BW, GK * BW)]], buf,
                              sem).wait()

    start_group(1 + GK, bbuf, bsem)

    cpv.wait()
    cpp.wait()

    def _unpack2(g32):
        return plsc.unpack(plsc.bitcast(g32, jnp.bfloat16),
                           format=plsc.PackFormat.INTERLEAVED)

    def pbody(g, c):
        riota = _iota16() + g * 16
        gsl = pl.ds(g * 16, 16)
        acc0 = jnp.zeros((16,), jnp.float32)
        acc1 = jnp.zeros((16,), jnp.float32)
        col = _iota16()
        for d in range(DIMH):
            u0, u1 = _unpack2(plsc.load_gather(abuf, [riota, col]))
            v0, v1 = _unpack2(plsc.load_gather(vrows, [riota, col]))
            acc0 = acc0 + u0 * v0
            acc1 = acc1 + u1 * v1
            col = (col + 1) & (DIMH - 1)
        mdotbuf[0, gsl] = acc0 + acc1
        return c

    lax.fori_loop(0, NG, pbody, 0)
    start_group(1, abuf, asem)

    def compute_group(kbase, buf):
        def gbody(g, c):
            riota = _iota16() + g * 16
            gsl = pl.ds(g * 16, 16)
            acca = [jnp.zeros((16,), jnp.float32) for _ in range(GK)]
            accb = [jnp.zeros((16,), jnp.float32) for _ in range(GK)]
            riotas = [riota + j * BW for j in range(GK)]
            col = _iota16()
            for d in range(DIMH):
                v0, v1 = _unpack2(plsc.load_gather(vrows, [riota, col]))
                for j in range(GK):
                    u0, u1 = _unpack2(plsc.load_gather(buf, [riotas[j], col]))
                    acca[j] = acca[j] + u0 * v0
                    accb[j] = accb[j] + u1 * v1
                col = (col + 1) & (DIMH - 1)
            for j in range(GK):
                mdotbuf[kbase + j, gsl] = ((acca[j] + accb[j]) *
                                           maskbuf[kbase + j, gsl])
            return c
        lax.fori_loop(0, NG, gbody, 0)

    def sbody(s2, c):
        ka = 2 * GK * s2 + 1
        wait_group(abuf, asem)
        compute_group(ka, abuf)

        @pl.when(ka + 2 * GK <= K - GK)
        def _():
            start_group(ka + 2 * GK, abuf, asem)

        kb = ka + GK
        wait_group(bbuf, bsem)
        compute_group(kb, bbuf)

        @pl.when(kb + 2 * GK <= K - GK)
        def _():
            start_group(kb + 2 * GK, bbuf, bsem)
        return c

    lax.fori_loop(0, NGRP // 2, sbody, 0)

    pltpu.sync_copy(mdotbuf, out.at[:, bsl])


@jax.jit
def _sc_dots(pair_u, pair_v, pos_u, neg_u, mask_pos, mask_neg,
             u_table, v_table):
    mesh = plsc.VectorSubcoreMesh(core_axis_name="c", subcore_axis_name="s")
    return pl.kernel(
        _sc_body,
        out_type=jax.ShapeDtypeStruct((K, B), jnp.float32),
        mesh=mesh,
        compiler_params=pltpu.CompilerParams(
            needs_layout_passes=False, use_tc_tiling_on_sc=False),
        scratch_types=[
            pltpu.VMEM((K * BW,), jnp.int32),
            pltpu.VMEM((K, BW), jnp.float32),
            pltpu.VMEM((BW,), jnp.int32),
            pltpu.VMEM((BW, KN), jnp.int32),
            pltpu.VMEM((BW, KN), jnp.int32),
            pltpu.VMEM((BW, KN), jnp.float32),
            pltpu.VMEM((BW, KN), jnp.float32),
            pltpu.VMEM((BW, DIMH), jnp.int32),
            pltpu.VMEM((K, BW), jnp.float32),
            pltpu.VMEM((GK * BW, DIMH), jnp.int32),
            pltpu.VMEM((GK * BW, DIMH), jnp.int32),
            pltpu.SemaphoreType.DMA,
            pltpu.SemaphoreType.DMA,
            pltpu.SemaphoreType.DMA,
            pltpu.SemaphoreType.DMA,
        ],
    )(pair_u, pair_v, pos_u, neg_u, mask_pos, mask_neg, u_table, v_table)


def _fin_body(dots_ref, out_ref):
    x = dots_ref[...]
    row = lax.broadcasted_iota(jnp.int32, (K, B), 0)
    s = jnp.where(row >= 1 + KN, -x, x)
    t = jnp.minimum(s, 0.0) - jnp.log1p(jnp.exp(-jnp.abs(s)))
    coef = jnp.where(row == 0, jnp.float32(1 + KN),
                     jnp.where(row >= 1 + KN, jnp.float32(1.0),
                               jnp.float32(-1.0)))
    out_ref[0, 0] = -jnp.sum(coef * t)


def _finish(dots):
    return pl.pallas_call(
        _fin_body,
        out_shape=jax.ShapeDtypeStruct((1, 1), jnp.float32),
        in_specs=[pl.BlockSpec(memory_space=pltpu.VMEM)],
        out_specs=pl.BlockSpec(memory_space=pltpu.SMEM),
    )(dots)


def kernel(pair_u, pair_v, pos_u, mask_pos_u, neg_u, mask_neg_u,
           u_table, v_table):
    def _pack_bf16(t):
        ti = lax.bitcast_convert_type(t, jnp.int32)
        r = (ti + 0x7FFF + ((ti >> 16) & 1)) >> 16
        return (r[:, 0::2] & 0xFFFF) | (r[:, 1::2] << 16)

    u_p = _pack_bf16(u_table)
    v_p = _pack_bf16(v_table)
    dots = _sc_dots(pair_u.astype(jnp.int32), pair_v.astype(jnp.int32),
                    pos_u.astype(jnp.int32), neg_u.astype(jnp.int32),
                    mask_pos_u, mask_neg_u, u_p, v_p)
    return _finish(dots)[0, 0]


# speedup vs baseline: 6.1008x; 6.1008x over previous
import jax
import jax.numpy as jnp
from jax import lax
from jax.experimental import pallas as pl
from jax.experimental.pallas import tpu as pltpu
from jax.experimental.pallas import tpu_sc as plsc

VOCAB = 100000
DIM = 64
DIMH = DIM // 2
B = 4096
KN = 20
K = 2 * KN + 1
NC = 2
NS = 16
NW = NC * NS
BW = B // NW
NG = BW // 16
GK = 5
NGRP = (K - 1) // GK


def _iota16():
    return lax.broadcasted_iota(jnp.int32, (16,), 0)


def _sc_body(pair_u, pair_v, pos_u, neg_u, mask_pos, mask_neg,
             u_table, v_table, out,
             idxbuf, maskbuf, pvbuf, ps, ns, mp, mn, vrows, mdotbuf,
             abuf, bbuf, semv, semp, asem, bsem):
    wid = lax.axis_index("s") * NC + lax.axis_index("c")
    base = wid * BW
    bsl = pl.ds(base, BW)

    pltpu.sync_copy(pair_u.at[bsl], idxbuf.at[pl.ds(0, BW)])
    pltpu.sync_copy(pair_v.at[bsl], pvbuf)
    cpv = pltpu.async_copy(v_table.at[pvbuf], vrows, semv)
    cpp = pltpu.async_copy(u_table.at[idxbuf.at[pl.ds(0, BW)]],
                           abuf.at[pl.ds(0, BW)], semp)
    pltpu.sync_copy(pos_u.at[bsl], ps)
    pltpu.sync_copy(neg_u.at[bsl], ns)
    pltpu.sync_copy(mask_pos.at[bsl], mp)
    pltpu.sync_copy(mask_neg.at[bsl], mn)

    for g in range(NG):
        maskbuf[0, pl.ds(g * 16, 16)] = jnp.full((16,), 1.0, jnp.float32)

    def tbody(k, c):
        kv = jnp.full((16,), 0, jnp.int32) + k
        for g in range(NG):
            biota = _iota16() + g * 16
            gsl = pl.ds(g * 16, 16)
            idxbuf[pl.ds((1 + k) * BW + g * 16, 16)] = plsc.load_gather(
                ps, [biota, kv])
            idxbuf[pl.ds((1 + KN + k) * BW + g * 16, 16)] = plsc.load_gather(
                ns, [biota, kv])
            maskbuf[1 + k, gsl] = plsc.load_gather(mp, [biota, kv])
            maskbuf[1 + KN + k, gsl] = plsc.load_gather(mn, [biota, kv])
        return c

    lax.fori_loop(0, KN, tbody, 0)

    def start_group(kbase, buf, sem):
        pltpu.async_copy(u_table.at[idxbuf.at[pl.ds(kbase * BW, GK * BW)]],
                         buf, sem)

    def wait_group(buf, sem):
        pltpu.make_async_copy(u_table.at[idxbuf.at[pl.ds(BW, GK * BW)]], buf,
                              sem).wait()

    start_group(1 + GK, bbuf, bsem)

    cpv.wait()
    cpp.wait()

    def _unpack2(g32):
        return plsc.unpack(plsc.bitcast(g32, jnp.bfloat16),
                           format=plsc.PackFormat.INTERLEAVED)

    def pbody(g, c):
        riota = _iota16() + g * 16
        gsl = pl.ds(g * 16, 16)
        acc0 = jnp.zeros((16,), jnp.float32)
        acc1 = jnp.zeros((16,), jnp.float32)
        col = _iota16()
        for d in range(DIMH):
            u0, u1 = _unpack2(plsc.load_gather(abuf, [riota, col]))
            v0, v1 = _unpack2(plsc.load_gather(vrows, [riota, col]))
            acc0 = acc0 + u0 * v0
            acc1 = acc1 + u1 * v1
            col = (col + 1) & (DIMH - 1)
        mdotbuf[0, gsl] = acc0 + acc1
        return c

    lax.fori_loop(0, NG, pbody, 0)
    start_group(1, abuf, asem)

    def compute_group(kbase, buf):
        def gbody(g, c):
            riota = _iota16() + g * 16
            gsl = pl.ds(g * 16, 16)
            acca = [jnp.zeros((16,), jnp.float32) for _ in range(GK)]
            accb = [jnp.zeros((16,), jnp.float32) for _ in range(GK)]
            riotas = [riota + j * BW for j in range(GK)]
            col = _iota16()
            for d in range(DIMH):
                v0, v1 = _unpack2(plsc.load_gather(vrows, [riota, col]))
                for j in range(GK):
                    u0, u1 = _unpack2(plsc.load_gather(buf, [riotas[j], col]))
                    acca[j] = acca[j] + u0 * v0
                    accb[j] = accb[j] + u1 * v1
                col = (col + 1) & (DIMH - 1)
            for j in range(GK):
                mdotbuf[kbase + j, gsl] = ((acca[j] + accb[j]) *
                                           maskbuf[kbase + j, gsl])
            return c
        lax.fori_loop(0, NG, gbody, 0)

    def sbody(s2, c):
        ka = 2 * GK * s2 + 1
        wait_group(abuf, asem)
        compute_group(ka, abuf)

        @pl.when(ka + 2 * GK <= K - GK)
        def _():
            start_group(ka + 2 * GK, abuf, asem)

        kb = ka + GK
        wait_group(bbuf, bsem)
        compute_group(kb, bbuf)

        @pl.when(kb + 2 * GK <= K - GK)
        def _():
            start_group(kb + 2 * GK, bbuf, bsem)
        return c

    lax.fori_loop(0, NGRP // 2, sbody, 0)

    pltpu.sync_copy(mdotbuf, out.at[:, bsl])


@jax.jit
def _sc_dots(pair_u, pair_v, pos_u, neg_u, mask_pos, mask_neg,
             u_table, v_table):
    mesh = plsc.VectorSubcoreMesh(core_axis_name="c", subcore_axis_name="s")
    return pl.kernel(
        _sc_body,
        out_type=jax.ShapeDtypeStruct((K, B), jnp.float32),
        mesh=mesh,
        compiler_params=pltpu.CompilerParams(
            needs_layout_passes=False, use_tc_tiling_on_sc=False),
        scratch_types=[
            pltpu.VMEM((K * BW,), jnp.int32),
            pltpu.VMEM((K, BW), jnp.float32),
            pltpu.VMEM((BW,), jnp.int32),
            pltpu.VMEM((BW, KN), jnp.int32),
            pltpu.VMEM((BW, KN), jnp.int32),
            pltpu.VMEM((BW, KN), jnp.float32),
            pltpu.VMEM((BW, KN), jnp.float32),
            pltpu.VMEM((BW, DIMH), jnp.int32),
            pltpu.VMEM((K, BW), jnp.float32),
            pltpu.VMEM((GK * BW, DIMH), jnp.int32),
            pltpu.VMEM((GK * BW, DIMH), jnp.int32),
            pltpu.SemaphoreType.DMA,
            pltpu.SemaphoreType.DMA,
            pltpu.SemaphoreType.DMA,
            pltpu.SemaphoreType.DMA,
        ],
    )(pair_u, pair_v, pos_u, neg_u, mask_pos, mask_neg, u_table, v_table)


def _fin_body(dots_ref, out_ref):
    x = dots_ref[...]
    row = lax.broadcasted_iota(jnp.int32, (K, B), 0)
    s = jnp.where(row >= 1 + KN, -x, x)
    t = jnp.minimum(s, 0.0) - jnp.log1p(jnp.exp(-jnp.abs(s)))
    coef = jnp.where(row == 0, jnp.float32(1 + KN),
                     jnp.where(row >= 1 + KN, jnp.float32(1.0),
                               jnp.float32(-1.0)))
    out_ref[0, 0] = -jnp.sum(coef * t)


def _finish(dots):
    return pl.pallas_call(
        _fin_body,
        out_shape=jax.ShapeDtypeStruct((1, 1), jnp.float32),
        in_specs=[pl.BlockSpec(memory_space=pltpu.VMEM)],
        out_specs=pl.BlockSpec(memory_space=pltpu.SMEM),
    )(dots)


def kernel(pair_u, pair_v, pos_u, mask_pos_u, neg_u, mask_neg_u,
           u_table, v_table):
    def _pack_bf16(t):
        ti = lax.bitcast_convert_type(t, jnp.int32)
        r = (ti + 0x7FFF + ((ti >> 16) & 1)) >> 16
        return (r[:, :DIMH] & 0xFFFF) | (r[:, DIMH:] << 16)

    u_p = _pack_bf16(u_table)
    v_p = _pack_bf16(v_table)
    dots = _sc_dots(pair_u.astype(jnp.int32), pair_v.astype(jnp.int32),
                    pos_u.astype(jnp.int32), neg_u.astype(jnp.int32),
                    mask_pos_u, mask_neg_u, u_p, v_p)
    return _finish(dots)[0, 0]


# speedup vs baseline: 11.5652x; 1.8957x over previous
import jax
import jax.numpy as jnp
from jax import lax
from jax.experimental import pallas as pl
from jax.experimental.pallas import tpu as pltpu
from jax.experimental.pallas import tpu_sc as plsc

VOCAB = 100000
DIM = 64
DIMH = DIM // 2
B = 4096
KN = 20
K = 2 * KN + 1
NC = 2
NS = 16
NW = NC * NS
BW = B // NW
BW2 = 2 * BW
NG = BW // 16
GK = 4
NGRP = (K - 1) // GK


def _iota16():
    return lax.broadcasted_iota(jnp.int32, (16,), 0)


def _sc_body(pair_u, pair_v, pos_u, neg_u, mask_pos, mask_neg,
             u_table, v_table, out,
             idxbuf, maskbuf, pvbuf, pv2, ps, ns, mp, mn, vrows, mdotbuf,
             abuf, bbuf, semv, semp, asem, bsem):
    wid = lax.axis_index("s") * NC + lax.axis_index("c")
    base = wid * BW
    bsl = pl.ds(base, BW)

    pltpu.sync_copy(pair_u.at[bsl], idxbuf.at[pl.ds(0, BW)])
    pltpu.sync_copy(pair_v.at[bsl], pvbuf)
    pltpu.sync_copy(pos_u.at[bsl], ps)
    pltpu.sync_copy(neg_u.at[bsl], ns)
    pltpu.sync_copy(mask_pos.at[bsl], mp)
    pltpu.sync_copy(mask_neg.at[bsl], mn)

    for g in range(NG):
        gsl = pl.ds(g * 16, 16)
        pu = idxbuf[gsl]
        idxbuf[gsl] = 2 * pu
        idxbuf[pl.ds(BW + g * 16, 16)] = 2 * pu + 1
        pv = pvbuf[gsl]
        pv2[gsl] = 2 * pv
        pv2[pl.ds(BW + g * 16, 16)] = 2 * pv + 1
        maskbuf[0, gsl] = jnp.full((16,), 1.0, jnp.float32)

    cpv = pltpu.async_copy(v_table.at[pv2], vrows, semv)
    cpp = pltpu.async_copy(u_table.at[idxbuf.at[pl.ds(0, BW2)]],
                           abuf.at[pl.ds(0, BW2)], semp)

    def tbody(k, c):
        kv = jnp.full((16,), 0, jnp.int32) + k
        for g in range(NG):
            biota = _iota16() + g * 16
            gsl = pl.ds(g * 16, 16)
            pi = plsc.load_gather(ps, [biota, kv])
            idxbuf[pl.ds((1 + k) * BW2 + g * 16, 16)] = 2 * pi
            idxbuf[pl.ds((1 + k) * BW2 + BW + g * 16, 16)] = 2 * pi + 1
            ni = plsc.load_gather(ns, [biota, kv])
            idxbuf[pl.ds((1 + KN + k) * BW2 + g * 16, 16)] = 2 * ni
            idxbuf[pl.ds((1 + KN + k) * BW2 + BW + g * 16, 16)] = 2 * ni + 1
            maskbuf[1 + k, gsl] = plsc.load_gather(mp, [biota, kv])
            maskbuf[1 + KN + k, gsl] = plsc.load_gather(mn, [biota, kv])
        return c

    lax.fori_loop(0, KN, tbody, 0)

    def start_group(kbase, buf, sem):
        pltpu.async_copy(u_table.at[idxbuf.at[pl.ds(kbase * BW2, GK * BW2)]],
                         buf, sem)

    def wait_group(buf, sem):
        pltpu.make_async_copy(u_table.at[idxbuf.at[pl.ds(BW2, GK * BW2)]],
                              buf, sem).wait()

    start_group(1 + GK, bbuf, bsem)

    cpv.wait()
    cpp.wait()

    def pbody(g, c):
        riota = _iota16() + g * 16
        gsl = pl.ds(g * 16, 16)
        acc0 = jnp.zeros((16,), jnp.float32)
        acc1 = jnp.zeros((16,), jnp.float32)
        col = _iota16()
        for d in range(DIMH):
            acc0 = acc0 + (plsc.load_gather(abuf, [riota, col]) *
                           plsc.load_gather(vrows, [riota, col]))
            acc1 = acc1 + (plsc.load_gather(abuf, [riota + BW, col]) *
                           plsc.load_gather(vrows, [riota + BW, col]))
            col = (col + 1) & (DIMH - 1)
        mdotbuf[0, gsl] = acc0 + acc1
        return c

    lax.fori_loop(0, NG, pbody, 0)
    start_group(1, abuf, asem)

    def compute_group(kbase, buf):
        def gbody(g, c):
            riota = _iota16() + g * 16
            gsl = pl.ds(g * 16, 16)
            acca = [jnp.zeros((16,), jnp.float32) for _ in range(GK)]
            accb = [jnp.zeros((16,), jnp.float32) for _ in range(GK)]
            rlo = [riota + j * BW2 for j in range(GK)]
            rhi = [riota + j * BW2 + BW for j in range(GK)]
            col = _iota16()
            for d in range(DIMH):
                vlo = plsc.load_gather(vrows, [riota, col])
                vhi = plsc.load_gather(vrows, [riota + BW, col])
                for j in range(GK):
                    acca[j] = acca[j] + plsc.load_gather(buf, [rlo[j], col]) * vlo
                    accb[j] = accb[j] + plsc.load_gather(buf, [rhi[j], col]) * vhi
                col = (col + 1) & (DIMH - 1)
            for j in range(GK):
                mdotbuf[kbase + j, gsl] = ((acca[j] + accb[j]) *
                                           maskbuf[kbase + j, gsl])
            return c
        lax.fori_loop(0, NG, gbody, 0)

    def sbody(s2, c):
        ka = 2 * GK * s2 + 1
        wait_group(abuf, asem)
        compute_group(ka, abuf)

        @pl.when(ka + 2 * GK <= K - GK)
        def _():
            start_group(ka + 2 * GK, abuf, asem)

        kb = ka + GK
        wait_group(bbuf, bsem)
        compute_group(kb, bbuf)

        @pl.when(kb + 2 * GK <= K - GK)
        def _():
            start_group(kb + 2 * GK, bbuf, bsem)
        return c

    lax.fori_loop(0, NGRP // 2, sbody, 0)

    pltpu.sync_copy(mdotbuf, out.at[:, bsl])


@jax.jit
def _sc_dots(pair_u, pair_v, pos_u, neg_u, mask_pos, mask_neg,
             u_table, v_table):
    mesh = plsc.VectorSubcoreMesh(core_axis_name="c", subcore_axis_name="s")
    return pl.kernel(
        _sc_body,
        out_type=jax.ShapeDtypeStruct((K, B), jnp.float32),
        mesh=mesh,
        compiler_params=pltpu.CompilerParams(
            needs_layout_passes=False, use_tc_tiling_on_sc=False),
        scratch_types=[
            pltpu.VMEM((K * BW2,), jnp.int32),
            pltpu.VMEM((K, BW), jnp.float32),
            pltpu.VMEM((BW,), jnp.int32),
            pltpu.VMEM((BW2,), jnp.int32),
            pltpu.VMEM((BW, KN), jnp.int32),
            pltpu.VMEM((BW, KN), jnp.int32),
            pltpu.VMEM((BW, KN), jnp.float32),
            pltpu.VMEM((BW, KN), jnp.float32),
            pltpu.VMEM((BW2, DIMH), jnp.float32),
            pltpu.VMEM((K, BW), jnp.float32),
            pltpu.VMEM((GK * BW2, DIMH), jnp.float32),
            pltpu.VMEM((GK * BW2, DIMH), jnp.float32),
            pltpu.SemaphoreType.DMA,
            pltpu.SemaphoreType.DMA,
            pltpu.SemaphoreType.DMA,
            pltpu.SemaphoreType.DMA,
        ],
    )(pair_u, pair_v, pos_u, neg_u, mask_pos, mask_neg, u_table, v_table)


def _fin_body(dots_ref, out_ref):
    x = dots_ref[...]
    row = lax.broadcasted_iota(jnp.int32, (K, B), 0)
    s = jnp.where(row >= 1 + KN, -x, x)
    t = jnp.minimum(s, 0.0) - jnp.log1p(jnp.exp(-jnp.abs(s)))
    coef = jnp.where(row == 0, jnp.float32(1 + KN),
                     jnp.where(row >= 1 + KN, jnp.float32(1.0),
                               jnp.float32(-1.0)))
    out_ref[0, 0] = -jnp.sum(coef * t)


def _finish(dots):
    return pl.pallas_call(
        _fin_body,
        out_shape=jax.ShapeDtypeStruct((1, 1), jnp.float32),
        in_specs=[pl.BlockSpec(memory_space=pltpu.VMEM)],
        out_specs=pl.BlockSpec(memory_space=pltpu.SMEM),
    )(dots)


def kernel(pair_u, pair_v, pos_u, mask_pos_u, neg_u, mask_neg_u,
           u_table, v_table):
    dots = _sc_dots(pair_u.astype(jnp.int32), pair_v.astype(jnp.int32),
                    pos_u.astype(jnp.int32), neg_u.astype(jnp.int32),
                    mask_pos_u, mask_neg_u,
                    u_table.reshape(2 * VOCAB, DIMH),
                    v_table.reshape(2 * VOCAB, DIMH))
    return _finish(dots)[0, 0]


# speedup vs baseline: 11.8948x; 1.0285x over previous
import jax
import jax.numpy as jnp
from jax import lax
from jax.experimental import pallas as pl
from jax.experimental.pallas import tpu as pltpu
from jax.experimental.pallas import tpu_sc as plsc

VOCAB = 100000
DIM = 64
B = 4096
KN = 20
K = 2 * KN + 1
NC = 2
NS = 16
NW = NC * NS
BW = B // NW
NG = BW // 16
GK = 5
NGRP = (K - 1) // GK


def _iota16():
    return lax.broadcasted_iota(jnp.int32, (16,), 0)


def _sc_body(pair_u, pair_v, pos_u, neg_u, mask_pos, mask_neg,
             u_table, v_table, out,
             idxbuf, maskbuf, pvbuf, ps, ns, mp, mn, vrows, mdotbuf,
             abuf, bbuf, semv, semp, asem, bsem):
    wid = lax.axis_index("s") * NC + lax.axis_index("c")
    base = wid * BW
    bsl = pl.ds(base, BW)

    pltpu.sync_copy(pair_u.at[bsl], idxbuf.at[pl.ds(0, BW)])
    pltpu.sync_copy(pair_v.at[bsl], pvbuf)
    cpv = pltpu.async_copy(v_table.at[pvbuf], vrows, semv)
    cpp = pltpu.async_copy(u_table.at[idxbuf.at[pl.ds(0, BW)]],
                           abuf.at[pl.ds(0, BW)], semp)
    pltpu.sync_copy(pos_u.at[bsl], ps)
    pltpu.sync_copy(neg_u.at[bsl], ns)
    pltpu.sync_copy(mask_pos.at[bsl], mp)
    pltpu.sync_copy(mask_neg.at[bsl], mn)

    for g in range(NG):
        maskbuf[0, pl.ds(g * 16, 16)] = jnp.full((16,), 1.0, jnp.float32)

    def tbody(k, c):
        kv = jnp.full((16,), 0, jnp.int32) + k
        for g in range(NG):
            biota = _iota16() + g * 16
            gsl = pl.ds(g * 16, 16)
            idxbuf[pl.ds((1 + k) * BW + g * 16, 16)] = plsc.load_gather(
                ps, [biota, kv])
            idxbuf[pl.ds((1 + KN + k) * BW + g * 16, 16)] = plsc.load_gather(
                ns, [biota, kv])
            maskbuf[1 + k, gsl] = plsc.load_gather(mp, [biota, kv])
            maskbuf[1 + KN + k, gsl] = plsc.load_gather(mn, [biota, kv])
        return c

    lax.fori_loop(0, KN, tbody, 0)

    def start_group(kbase, buf, sem):
        pltpu.async_copy(u_table.at[idxbuf.at[pl.ds(kbase * BW, GK * BW)]],
                         buf, sem)

    def wait_group(buf, sem):
        pltpu.make_async_copy(u_table.at[idxbuf.at[pl.ds(BW, GK * BW)]], buf,
                              sem).wait()

    start_group(1 + GK, bbuf, bsem)

    cpv.wait()
    cpp.wait()

    def pbody(g, c):
        riota = _iota16() + g * 16
        gsl = pl.ds(g * 16, 16)
        acc0 = jnp.zeros((16,), jnp.float32)
        acc1 = jnp.zeros((16,), jnp.float32)
        col = _iota16()
        for d in range(0, DIM, 2):
            acc0 = acc0 + (plsc.load_gather(abuf, [riota, col]) *
                           plsc.load_gather(vrows, [riota, col]))
            col1 = (col + 1) & (DIM - 1)
            acc1 = acc1 + (plsc.load_gather(abuf, [riota, col1]) *
                           plsc.load_gather(vrows, [riota, col1]))
            col = (col1 + 1) & (DIM - 1)
        mdotbuf[0, gsl] = acc0 + acc1
        return c

    lax.fori_loop(0, NG, pbody, 0)
    start_group(1, abuf, asem)

    def compute_group(kbase, buf):
        def gbody(g, c):
            riota = _iota16() + g * 16
            gsl = pl.ds(g * 16, 16)
            accs = [jnp.zeros((16,), jnp.float32) for _ in range(GK)]
            riotas = [riota + j * BW for j in range(GK)]
            col = _iota16()
            for d in range(DIM):
                vc = plsc.load_gather(vrows, [riota, col])
                for j in range(GK):
                    accs[j] = accs[j] + plsc.load_gather(buf, [riotas[j], col]) * vc
                col = (col + 1) & (DIM - 1)
            for j in range(GK):
                mdotbuf[kbase + j, gsl] = accs[j] * maskbuf[kbase + j, gsl]
            return c
        lax.fori_loop(0, NG, gbody, 0)

    def sbody(s2, c):
        ka = 2 * GK * s2 + 1
        wait_group(abuf, asem)
        compute_group(ka, abuf)

        @pl.when(ka + 2 * GK <= K - GK)
        def _():
            start_group(ka + 2 * GK, abuf, asem)

        kb = ka + GK
        wait_group(bbuf, bsem)
        compute_group(kb, bbuf)

        @pl.when(kb + 2 * GK <= K - GK)
        def _():
            start_group(kb + 2 * GK, bbuf, bsem)
        return c

    lax.fori_loop(0, NGRP // 2, sbody, 0)

    pltpu.sync_copy(mdotbuf, out.at[:, bsl])


@jax.jit
def _sc_dots(pair_u, pair_v, pos_u, neg_u, mask_pos, mask_neg,
             u_table, v_table):
    mesh = plsc.VectorSubcoreMesh(core_axis_name="c", subcore_axis_name="s")
    return pl.kernel(
        _sc_body,
        out_type=jax.ShapeDtypeStruct((K, B), jnp.float32),
        mesh=mesh,
        compiler_params=pltpu.CompilerParams(
            needs_layout_passes=False, use_tc_tiling_on_sc=False),
        scratch_types=[
            pltpu.VMEM((K * BW,), jnp.int32),
            pltpu.VMEM((K, BW), jnp.float32),
            pltpu.VMEM((BW,), jnp.int32),
            pltpu.VMEM((BW, KN), jnp.int32),
            pltpu.VMEM((BW, KN), jnp.int32),
            pltpu.VMEM((BW, KN), jnp.float32),
            pltpu.VMEM((BW, KN), jnp.float32),
            pltpu.VMEM((BW, DIM), jnp.float32),
            pltpu.VMEM((K, BW), jnp.float32),
            pltpu.VMEM((GK * BW, DIM), jnp.float32),
            pltpu.VMEM((GK * BW, DIM), jnp.float32),
            pltpu.SemaphoreType.DMA,
            pltpu.SemaphoreType.DMA,
            pltpu.SemaphoreType.DMA,
            pltpu.SemaphoreType.DMA,
        ],
    )(pair_u, pair_v, pos_u, neg_u, mask_pos, mask_neg, u_table, v_table)


def _fin_body(dots_ref, out_ref):
    x = dots_ref[...]
    row = lax.broadcasted_iota(jnp.int32, (K, B), 0)
    s = jnp.where(row >= 1 + KN, -x, x)
    t = jnp.minimum(s, 0.0) - jnp.log1p(jnp.exp(-jnp.abs(s)))
    coef = jnp.where(row == 0, jnp.float32(1 + KN),
                     jnp.where(row >= 1 + KN, jnp.float32(1.0),
                               jnp.float32(-1.0)))
    out_ref[0, 0] = -jnp.sum(coef * t)


def _finish(dots):
    return pl.pallas_call(
        _fin_body,
        out_shape=jax.ShapeDtypeStruct((1, 1), jnp.float32),
        in_specs=[pl.BlockSpec(memory_space=pltpu.VMEM)],
        out_specs=pl.BlockSpec(memory_space=pltpu.SMEM),
    )(dots)


def kernel(pair_u, pair_v, pos_u, mask_pos_u, neg_u, mask_neg_u,
           u_table, v_table):
    dots = _sc_dots(pair_u.astype(jnp.int32), pair_v.astype(jnp.int32),
                    pos_u.astype(jnp.int32), neg_u.astype(jnp.int32),
                    mask_pos_u, mask_neg_u, u_table, v_table)
    return _finish(dots)[0, 0]
